# baseline (speedup 1.0000x reference)
import functools

import jax
import jax.numpy as jnp
from jax import lax
from jax.experimental import pallas as pl
from jax.experimental.pallas import tpu as pltpu
from jax.experimental.pallas import tpu_sc as plsc

_EMB_DIMS = (32, 32, 64, 32, 32)
_OFFS = (0, 32, 64, 128, 160)
_TOTAL = 192
_MODEL = 768
_LG = 128
_BLK = 2048


def _gather_concat(x3, t0, t1, t2, t3, t4):
    n_groups = x3.shape[1]
    n_tok = n_groups * _LG
    info = plsc.get_sparse_core_info()
    nc = info.num_cores
    nw = nc * info.num_subcores
    g_per_w = n_groups // nw
    mesh = plsc.VectorSubcoreMesh(core_axis_name="c", subcore_axis_name="s")

    scratch = (
        [pltpu.VMEM((g_per_w, _LG), jnp.int32) for _ in range(5)]
        + [pltpu.VMEM((_LG, d), jnp.float32) for d in _EMB_DIMS]
        + [pltpu.VMEM((_LG, d), jnp.float32) for d in _EMB_DIMS]
        + [pltpu.SemaphoreType.DMA,
           pltpu.SemaphoreType.DMA,
           pltpu.SemaphoreType.DMA]
    )

    @functools.partial(
        pl.kernel,
        mesh=mesh,
        out_type=jax.ShapeDtypeStruct((n_tok, _TOTAL), jnp.float32),
        scratch_types=scratch,
        compiler_params=pltpu.CompilerParams(use_tc_tiling_on_sc=False),
    )
    def k(x_ref, r0, r1, r2, r3, r4, h_ref,
          i0, i1, i2, i3, i4,
          a0, a1, a2, a3, a4,
          b0, b1, b2, b3, b4,
          sg, ss0, ss1):
        tbls = (r0, r1, r2, r3, r4)
        idxs = (i0, i1, i2, i3, i4)
        rows = ((a0, a1, a2, a3, a4), (b0, b1, b2, b3, b4))
        ssems = (ss0, ss1)
        wid = lax.axis_index("s") * nc + lax.axis_index("c")
        g0 = wid * g_per_w
        for f in range(5):
            pltpu.sync_copy(x_ref.at[f, pl.ds(g0, g_per_w)], idxs[f])
        pending = [None, None]
        for it in range(g_per_w):
            s = it % 2
            if pending[s] is not None:
                for cp in pending[s]:
                    cp.wait()
            gathers = [
                pltpu.async_copy(tbls[f].at[idxs[f].at[it]], rows[s][f], sg)
                for f in range(5)
            ]
            for cp in gathers:
                cp.wait()
            row0 = (g0 + it) * _LG
            pending[s] = [
                pltpu.async_copy(
                    rows[s][f],
                    h_ref.at[pl.ds(row0, _LG), pl.ds(_OFFS[f], _EMB_DIMS[f])],
                    ssems[s])
                for f in range(5)
            ]
        for s in range(2):
            if pending[s] is not None:
                for cp in pending[s]:
                    cp.wait()

    return k(x3, t0, t1, t2, t3, t4)


def _mm_body(h_ref, w_ref, b_ref, o_ref):
    o_ref[...] = (
        jnp.dot(h_ref[...], w_ref[...], preferred_element_type=jnp.float32)
        + b_ref[...]
    )


def _mm_body_alias(h_ref, w_ref, b_ref, prev_ref, o_ref):
    del prev_ref
    _mm_body(h_ref, w_ref, b_ref, o_ref)


def _encode_chunk(h_b, enc_w, enc_b2, n_tok_total, blk0, out_prev):
    nblk = h_b.shape[0] // _BLK
    h_spec = pl.BlockSpec((_BLK, _TOTAL), lambda i: (i, 0))
    w_spec = pl.BlockSpec((_TOTAL, _MODEL), lambda i: (0, 0))
    b_spec = pl.BlockSpec((1, _MODEL), lambda i: (0, 0))
    out_spec = pl.BlockSpec(
        (_BLK, _MODEL), lambda i, *, b0=blk0: (b0 + i, 0))
    out_shape = jax.ShapeDtypeStruct((n_tok_total, _MODEL), jnp.float32)
    if out_prev is None:
        return pl.pallas_call(
            _mm_body,
            grid=(nblk,),
            in_specs=[h_spec, w_spec, b_spec],
            out_specs=out_spec,
            out_shape=out_shape,
        )(h_b, enc_w, enc_b2)
    return pl.pallas_call(
        _mm_body_alias,
        grid=(nblk,),
        in_specs=[h_spec, w_spec, b_spec,
                  pl.BlockSpec(memory_space=pl.ANY)],
        out_specs=out_spec,
        out_shape=out_shape,
        input_output_aliases={3: 0},
    )(h_b, enc_w, enc_b2, out_prev)


def kernel(x, table_0, table_1, table_2, table_3, table_4, enc_w, enc_b):
    b, s, f = x.shape
    n_tok = b * s
    tables = (table_0, table_1, table_2, table_3, table_4)
    xi = x.astype(jnp.int32)
    enc_b2 = enc_b.reshape(1, _MODEL)
    hs = []
    for bi in range(b):
        x3 = xi[bi].reshape(s // _LG, _LG, f).transpose(2, 0, 1)
        hs.append(_gather_concat(x3, *tables))
    out = None
    for bi in range(b):
        out = _encode_chunk(
            hs[bi], enc_w, enc_b2, n_tok, bi * (s // _BLK), out)
    return out.reshape(b, s, _MODEL)


# speedup vs baseline: 1.3602x; 1.3602x over previous
import functools

import jax
import jax.numpy as jnp
from jax import lax
from jax.experimental import pallas as pl
from jax.experimental.pallas import tpu as pltpu
from jax.experimental.pallas import tpu_sc as plsc

_EMB_DIMS = (32, 32, 64, 32, 32)
_PACK_OFF = (0, 32, 64, 0, 32)
_TOTAL = 192
_MODEL = 768
_LG = 128
_BLK = 4096


def _gather_pack(x3, t0, t1, t2, t3, t4):
    n_groups = x3.shape[1]
    n_tok = n_groups * _LG
    info = plsc.get_sparse_core_info()
    nc = info.num_cores
    nw = nc * info.num_subcores
    g_per_w = n_groups // nw
    mesh = plsc.VectorSubcoreMesh(core_axis_name="c", subcore_axis_name="s")

    scratch = (
        [pltpu.VMEM((g_per_w, _LG), jnp.int32) for _ in range(5)]
        + [pltpu.VMEM((_LG, d), jnp.float32) for d in _EMB_DIMS]
        + [pltpu.VMEM((_LG, d), jnp.float32) for d in _EMB_DIMS]
        + [pltpu.VMEM((_LG, 64), jnp.float32)]
        + [pltpu.SemaphoreType.DMA,
           pltpu.SemaphoreType.DMA,
           pltpu.SemaphoreType.DMA]
    )

    @functools.partial(
        pl.kernel,
        mesh=mesh,
        out_type=jax.ShapeDtypeStruct((2 * n_tok, _LG), jnp.float32),
        scratch_types=scratch,
        compiler_params=pltpu.CompilerParams(use_tc_tiling_on_sc=False),
    )
    def k(x_ref, r0, r1, r2, r3, r4, h_ref,
          i0, i1, i2, i3, i4,
          a0, a1, a2, a3, a4,
          b0, b1, b2, b3, b4,
          zbuf, sg, ss0, ss1):
        tbls = (r0, r1, r2, r3, r4)
        idxs = (i0, i1, i2, i3, i4)
        rows = ((a0, a1, a2, a3, a4), (b0, b1, b2, b3, b4))
        ssems = (ss0, ss1)
        wid = lax.axis_index("s") * nc + lax.axis_index("c")
        g0 = wid * g_per_w

        def zrow(t, _):
            for j in range(4):
                zbuf[t, pl.ds(16 * j, 16)] = jnp.zeros((16,), jnp.float32)
            return _
        lax.fori_loop(0, _LG, zrow, 0)

        for f in range(5):
            pltpu.sync_copy(x_ref.at[f, pl.ds(g0, g_per_w)], idxs[f])
        pending = [None, None]
        for it in range(g_per_w):
            s = it % 2
            if pending[s] is not None:
                for cp in pending[s]:
                    cp.wait()
            gathers = [
                pltpu.async_copy(tbls[f].at[idxs[f].at[it]], rows[s][f], sg)
                for f in range(5)
            ]
            for cp in gathers:
                cp.wait()
            row_l = (g0 + it) * _LG
            row_r = n_tok + row_l
            base = (row_l, row_l, row_l, row_r, row_r)
            pending[s] = [
                pltpu.async_copy(
                    rows[s][f],
                    h_ref.at[pl.ds(base[f], _LG),
                             pl.ds(_PACK_OFF[f], _EMB_DIMS[f])],
                    ssems[s])
                for f in range(5)
            ]
            pending[s].append(
                pltpu.async_copy(
                    zbuf, h_ref.at[pl.ds(row_r, _LG), pl.ds(64, 64)],
                    ssems[s]))
        for s in range(2):
            if pending[s] is not None:
                for cp in pending[s]:
                    cp.wait()

    return k(x3, t0, t1, t2, t3, t4)


def _encode(h2, w_l, w_r, enc_b2, n_tok):
    nblk = n_tok // _BLK

    def body(hl_ref, hr_ref, wl_ref, wr_ref, b_ref, o_ref):
        o_ref[...] = (
            jnp.dot(hl_ref[...], wl_ref[...],
                    preferred_element_type=jnp.float32)
            + jnp.dot(hr_ref[...], wr_ref[...],
                      preferred_element_type=jnp.float32)
            + b_ref[...]
        )

    return pl.pallas_call(
        body,
        grid=(nblk,),
        in_specs=[
            pl.BlockSpec((_BLK, _LG), lambda i: (i, 0)),
            pl.BlockSpec((_BLK, _LG), lambda i, *, nb=nblk: (nb + i, 0)),
            pl.BlockSpec((_LG, _MODEL), lambda i: (0, 0)),
            pl.BlockSpec((_LG, _MODEL), lambda i: (0, 0)),
            pl.BlockSpec((1, _MODEL), lambda i: (0, 0)),
        ],
        out_specs=pl.BlockSpec((_BLK, _MODEL), lambda i: (i, 0)),
        out_shape=jax.ShapeDtypeStruct((n_tok, _MODEL), jnp.float32),
    )(h2, h2, w_l, w_r, enc_b2)


def kernel(x, table_0, table_1, table_2, table_3, table_4, enc_w, enc_b):
    b, s, f = x.shape
    n_tok = b * s
    xi = x.astype(jnp.int32)
    x3 = xi.reshape(n_tok // _LG, _LG, f).transpose(2, 0, 1)
    h2 = _gather_pack(x3, table_0, table_1, table_2, table_3, table_4)
    w_l = enc_w[:_LG]
    w_r = jnp.pad(enc_w[_LG:], ((0, 2 * _LG - _TOTAL), (0, 0)))
    out = _encode(h2, w_l, w_r, enc_b.reshape(1, _MODEL), n_tok)
    return out.reshape(b, s, _MODEL)
